# baseline (speedup 1.0000x reference)
import functools

import jax
import jax.numpy as jnp
from jax import lax
from jax.experimental import pallas as pl
from jax.experimental.pallas import tpu as pltpu
from jax.experimental.pallas import tpu_sc as plsc

NUM_EMB_ROWS = 1_000_000
NUM_IDX = 819_200
NC = 2
NS = 16
NW = NC * NS
LANES = 16

NEP = 1 << 20
IDX_COLS = 128
IDX_ROWS_TOTAL = NUM_IDX // IDX_COLS
ROWS_PER_T = IDX_ROWS_TOTAL // NW
IDXB = 8
NIDXB = ROWS_PER_T // IDXB
SP_SLICE = NEP // NS
ZB = 4096

TROWS = 62_496
RCHUNK = 2_080
NFULL = TROWS // RCHUNK
RTAIL = TROWS - NFULL * RCHUNK
REXTRA = NUM_EMB_ROWS - NS * TROWS


def _fused_call(ebi2d, wcols):
    mesh = plsc.VectorSubcoreMesh(core_axis_name="c", subcore_axis_name="s")

    @functools.partial(
        pl.kernel,
        out_type=jax.ShapeDtypeStruct((NW * 6 * LANES,), jnp.float32),
        mesh=mesh,
        scratch_types=[
            pltpu.VMEM_SHARED((NEP,), jnp.float32),
            [pltpu.VMEM((IDXB, IDX_COLS), jnp.int32) for _ in range(2)],
            pltpu.VMEM((IDX_COLS,), jnp.float32),
            pltpu.VMEM((ZB,), jnp.float32),
            [[pltpu.VMEM((RCHUNK,), jnp.float32) for _ in range(13)]
             for _ in range(2)],
            pltpu.VMEM((6 * LANES,), jnp.float32),
            pltpu.SemaphoreType.DMA,
            [pltpu.SemaphoreType.DMA for _ in range(2)],
        ],
        compiler_params=pltpu.CompilerParams(needs_layout_passes=False),
    )
    def fused_kernel(ebi_hbm, *rest):
        wc_hbm = rest[:12]
        out_hbm = rest[12]
        hist_sp, idxb, ones_v, zbuf = rest[13:17]
        bufs = rest[17]
        ob = rest[18]
        scat_sem = rest[19]
        sems = rest[20]
        c = lax.axis_index("c")
        s = lax.axis_index("s")
        wid = s * NC + c

        def fill_z(i, _):
            zbuf[pl.ds(i * LANES, LANES)] = jnp.zeros((LANES,), jnp.float32)
            return 0

        lax.fori_loop(0, ZB // LANES, fill_z, 0)

        def fill_o(i, _):
            ones_v[pl.ds(i * LANES, LANES)] = jnp.ones((LANES,), jnp.float32)
            return 0

        lax.fori_loop(0, IDX_COLS // LANES, fill_o, 0)

        base_sp = s * SP_SLICE

        def zero_sp(i, _):
            pltpu.sync_copy(zbuf, hist_sp.at[pl.ds(base_sp + i * ZB, ZB)])
            return 0

        lax.fori_loop(0, SP_SLICE // ZB, zero_sp, 0)
        plsc.subcore_barrier()

        row0 = c * (IDX_ROWS_TOTAL // NC) + s * ROWS_PER_T

        rbase = s * TROWS
        gate = jnp.where(s == NS - 1, 1.0, 0.0).astype(jnp.float32)
        stages = [(rbase + k * RCHUNK, RCHUNK, None) for k in range(NFULL)]
        stages.append((rbase + NFULL * RCHUNK, RTAIL, None))
        stages.append((NUM_EMB_ROWS - REXTRA, REXTRA, gate))

        def fire_w(sidx, ro, nr):
            return [pltpu.async_copy(wc_hbm[i].at[pl.ds(ro, nr)],
                                     bufs[sidx][i].at[pl.ds(0, nr)],
                                     sems[sidx])
                    for i in range(12)]

        def stage_h(sidx, ro, nr):
            pltpu.sync_copy(hist_sp.at[pl.ds(ro, nr)],
                            bufs[sidx][12].at[pl.ds(0, nr)])

        chunk_descs = [None] * NIDXB
        for k in range(NIDXB):
            b = k % 2
            if k >= 2:
                for d in chunk_descs[k - 2]:
                    d.wait()
            pltpu.sync_copy(ebi_hbm.at[pl.ds(row0 + k * IDXB, IDXB)], idxb[b])
            chunk_descs[k] = [
                pltpu.async_copy(ones_v, hist_sp.at[idxb[b].at[j]],
                                 scat_sem, add=True)
                for j in range(IDXB)]
        for k in (NIDXB - 2, NIDXB - 1):
            for d in chunk_descs[k]:
                d.wait()
        plsc.subcore_barrier()

        def rowgroup_body(sidx, gv):
            wcb = bufs[sidx]

            def body(t, accs):
                a = list(accs)
                sl = pl.ds(t * LANES, LANES)
                h = wcb[12][sl]
                if gv is not None:
                    h = h * gv
                for cc in range(3):
                    a[cc] = a[cc] + h * (wcb[0 + cc][sl] + wcb[6 + cc][sl])
                    a[3 + cc] = a[3 + cc] + h * (wcb[3 + cc][sl] + wcb[9 + cc][sl])
                return tuple(a)
            return body

        accs = (jnp.zeros((LANES,), jnp.float32),) * 6
        descs = fire_w(0, stages[0][0], stages[0][1])
        stage_h(0, stages[0][0], stages[0][1])
        for i, (ro, nr, g) in enumerate(stages):
            sidx = i % 2
            nxt = None
            if i + 1 < len(stages):
                nro, nnr, _ = stages[i + 1]
                nxt = fire_w(1 - sidx, nro, nnr)
                stage_h(1 - sidx, nro, nnr)
            for d in descs:
                d.wait()
            accs = lax.fori_loop(0, nr // LANES, rowgroup_body(sidx, g), accs)
            descs = nxt

        for i in range(6):
            ob[pl.ds(i * LANES, LANES)] = accs[i]
        pltpu.sync_copy(ob, out_hbm.at[pl.ds(wid * 6 * LANES, 6 * LANES)])

    return fused_kernel(ebi2d, *wcols)


def kernel(eb_input, eb_offset, W0, W1, W2, W3):
    del eb_offset
    ebi2d = eb_input.reshape(IDX_ROWS_TOTAL, IDX_COLS)
    wcols = [W[:, cc] for W in (W0, W1, W2, W3) for cc in range(3)]
    partials = _fused_call(ebi2d, wcols)
    return jnp.sum(partials.reshape(NW, 6, LANES), axis=(0, 2))
